# baseline (speedup 1.0000x reference)
import functools

import jax
import jax.numpy as jnp
from jax import lax
from jax.experimental import pallas as pl
from jax.experimental.pallas import tpu as pltpu

N = 10000
DIN = 256
DH = 256
DT = 100
DNF = 512
BLK = 2000

_DN = (((1,), (1,)), ((), ()))
_DNT = (((0,), (1,)), ((), ()))


def _fused(tsb_ref, x_ref, mem_ref, h_ref,
           tw_ref, tb_ref, wih_ref, whh_ref, wm_ref,
           bih_ref, bhh_ref, bmap_ref, out_ref):
    dtrow = tsb_ref[0, 0] - tsb_ref[1, 0]
    twc = jnp.reshape(tw_ref[...], (DT, 1))
    tbc = jnp.reshape(tb_ref[...], (DT, 1))
    arg = twc * dtrow + tbc
    y = arg * 0.15915494309189535
    fr = y - jnp.round(y)
    s = fr * fr
    tfT = (0.9999994436793983
           + s * (-19.739034372931126
                  + s * (64.93061336990448
                         + s * (-85.29597096153826
                                + s * (58.912555324414804
                                       + s * -21.28302159300549)))))
    x = x_ref[...]
    mem = mem_ref[...]
    gi = (lax.dot_general(x, wih_ref[:, 0:DIN], _DN,
                          preferred_element_type=jnp.float32)
          + lax.dot_general(tfT, wih_ref[:, DIN:DIN + DT], _DNT,
                            preferred_element_type=jnp.float32))
    gh = lax.dot_general(mem, whh_ref[...], _DN,
                         preferred_element_type=jnp.float32)
    gi = gi + bih_ref[...]
    gh = gh + bhh_ref[...]
    r = 0.5 + 0.5 * jnp.tanh(0.5 * (gi[:, 0:DH] + gh[:, 0:DH]))
    z = 0.5 + 0.5 * jnp.tanh(0.5 * (gi[:, DH:2 * DH] + gh[:, DH:2 * DH]))
    n = jnp.tanh(gi[:, 2 * DH:3 * DH] + r * gh[:, 2 * DH:3 * DH])
    memory = (1.0 - z) * n + z * mem
    out_ref[...] = (memory
                    + lax.dot_general(h_ref[...], wm_ref[...], _DN,
                                      preferred_element_type=jnp.float32)
                    + bmap_ref[...])


@functools.partial(jax.jit, static_argnames=("interpret",))
def _run(mem_input, mem, ts, mem_ts, h, time_w, time_b,
         W_ih, W_hh, b_ih, b_hh, W_map, b_map, interpret=False):
    grid = (N // BLK,)
    row = lambda i: (i, 0)
    rep = lambda i: (0, 0)
    return pl.pallas_call(
        _fused,
        grid=grid,
        in_specs=[
            pl.BlockSpec((2, 1, 1, BLK), lambda i: (0, i, 0, 0)),
            pl.BlockSpec((BLK, DIN), row),
            pl.BlockSpec((BLK, DH), row),
            pl.BlockSpec((BLK, DNF), row),
            pl.BlockSpec((DT,), lambda i: (0,)),
            pl.BlockSpec((DT,), lambda i: (0,)),
            pl.BlockSpec((3 * DH, DIN + DT), rep),
            pl.BlockSpec((3 * DH, DH), rep),
            pl.BlockSpec((DH, DNF), rep),
            pl.BlockSpec((3 * DH,), lambda i: (0,)),
            pl.BlockSpec((3 * DH,), lambda i: (0,)),
            pl.BlockSpec((DH,), lambda i: (0,)),
        ],
        out_specs=pl.BlockSpec((BLK, DH), row),
        out_shape=jax.ShapeDtypeStruct((N, DH), jnp.float32),
        compiler_params=pltpu.CompilerParams(
            dimension_semantics=("parallel",),
            allow_input_fusion=[True] + [False] * 12),
        interpret=interpret,
    )(jnp.concatenate([ts, mem_ts]).reshape(2, N // BLK, 1, BLK),
      mem_input, mem, h,
      time_w, time_b, W_ih, W_hh, W_map, b_ih, b_hh, b_map)


def kernel(mem_input, mem, ts, mem_ts, h, time_w, time_b,
           W_ih, W_hh, b_ih, b_hh, W_map, b_map):
    return _run(mem_input, mem, ts, mem_ts, h, time_w, time_b,
                W_ih, W_hh, b_ih, b_hh, W_map, b_map)


# speedup vs baseline: 1.0058x; 1.0058x over previous
import jax
import jax.numpy as jnp
from jax import lax
from jax.experimental import pallas as pl
from jax.experimental.pallas import tpu as pltpu

N = 10000
DIN = 256
DH = 256
DT = 100
DNF = 512
BLK = 2000

_DN = (((1,), (1,)), ((), ()))
_DNT = (((0,), (1,)), ((), ()))


def _fused(tsb_ref, x_ref, mem_ref, h_ref,
           tw_ref, tb_ref, wih_ref, whh_ref, wm_ref,
           bih_ref, bhh_ref, bmap_ref, out_ref):
    dtrow = tsb_ref[0, 0] - tsb_ref[1, 0]
    twc = jnp.reshape(tw_ref[...], (DT, 1))
    tbc = jnp.reshape(tb_ref[...], (DT, 1))
    arg = twc * dtrow + tbc
    y = arg * 0.15915494309189535
    fr = y - jnp.round(y)
    s = fr * fr
    tfT = (0.9999994436793983
           + s * (-19.739034372931126
                  + s * (64.93061336990448
                         + s * (-85.29597096153826
                                + s * (58.912555324414804
                                       + s * -21.28302159300549)))))
    x = x_ref[...]
    mem = mem_ref[...]
    gi = (lax.dot_general(x, wih_ref[:, 0:DIN], _DN,
                          preferred_element_type=jnp.float32)
          + lax.dot_general(tfT, wih_ref[:, DIN:DIN + DT], _DNT,
                            preferred_element_type=jnp.float32))
    gh = lax.dot_general(mem, whh_ref[...], _DN,
                         preferred_element_type=jnp.float32)
    gi = gi + bih_ref[...]
    gh = gh + bhh_ref[...]
    r = 0.5 + 0.5 * jnp.tanh(0.5 * (gi[:, 0:DH] + gh[:, 0:DH]))
    z = 0.5 + 0.5 * jnp.tanh(0.5 * (gi[:, DH:2 * DH] + gh[:, DH:2 * DH]))
    n = jnp.tanh(gi[:, 2 * DH:3 * DH] + r * gh[:, 2 * DH:3 * DH])
    memory = (1.0 - z) * n + z * mem
    out_ref[...] = (memory
                    + lax.dot_general(h_ref[...], wm_ref[...], _DN,
                                      preferred_element_type=jnp.float32)
                    + bmap_ref[...])


@jax.jit
def _run(mem_input, mem, ts, mem_ts, h, time_w, time_b,
         W_ih, W_hh, b_ih, b_hh, W_map, b_map):
    grid = (N // BLK,)
    row = lambda i: (i, 0)
    rep = lambda i: (0, 0)
    return pl.pallas_call(
        _fused,
        grid=grid,
        in_specs=[
            pl.BlockSpec((2, 1, 1, BLK), lambda i: (0, i, 0, 0)),
            pl.BlockSpec((BLK, DIN), row),
            pl.BlockSpec((BLK, DH), row),
            pl.BlockSpec((BLK, DNF), row),
            pl.BlockSpec((DT,), lambda i: (0,)),
            pl.BlockSpec((DT,), lambda i: (0,)),
            pl.BlockSpec((3 * DH, DIN + DT), rep),
            pl.BlockSpec((3 * DH, DH), rep),
            pl.BlockSpec((DH, DNF), rep),
            pl.BlockSpec((3 * DH,), lambda i: (0,)),
            pl.BlockSpec((3 * DH,), lambda i: (0,)),
            pl.BlockSpec((DH,), lambda i: (0,)),
        ],
        out_specs=pl.BlockSpec((BLK, DH), row),
        out_shape=jax.ShapeDtypeStruct((N, DH), jnp.float32),
        compiler_params=pltpu.CompilerParams(
            dimension_semantics=("parallel",),
            allow_input_fusion=[True] + [False] * 12),
    )(jnp.concatenate([ts, mem_ts]).reshape(2, N // BLK, 1, BLK),
      mem_input, mem, h,
      time_w, time_b, W_ih, W_hh, W_map, b_ih, b_hh, b_map)


def kernel(mem_input, mem, ts, mem_ts, h, time_w, time_b,
           W_ih, W_hh, b_ih, b_hh, W_map, b_map):
    return _run(mem_input, mem, ts, mem_ts, h, time_w, time_b,
                W_ih, W_hh, b_ih, b_hh, W_map, b_map)
